# baseline (speedup 1.0000x reference)
import functools

import jax
import jax.numpy as jnp
from jax import lax
from jax.experimental import pallas as pl
from jax.experimental.pallas import tpu as pltpu
from jax.experimental.pallas import tpu_sc as plsc

N = 10000
E = 320000
C = 128
P = 16

NC = 2
NS = 16
NW = NC * NS
H = 48
EPW = 10080
NCHUNK = EPW // (2 * H)
IBLK = 15
NBLK = NCHUNK // IBLK
IROWS = 2 * IBLK
EPAD = NW * EPW - E
NPAD = 10240
DUMP = 10100
ROWS_PER_SUB = NPAD // NS
ZROWS = 32
DROWS = 79

f32 = jnp.float32
bf16 = jnp.bfloat16
i32 = jnp.int32


def _edge_body(qs_hbm, kv_hbm, src_hbm, dst_hbm,
               num_out, den_out,
               qab0, qab1, kvb0, kvb1, wvb, srcb, dstb, zbuf, denv,
               num_sh, sq0, sq1, sk0, sk1):
    cid = lax.axis_index("c")
    sid = lax.axis_index("s")
    wid = sid * NC + cid

    zerov = jnp.zeros((16,), f32)
    lane0 = lax.iota(i32, 16) == 0
    himask = jnp.full((16,), -65536, i32)
    dn = lax.GatherDimensionNumbers(offset_dims=(), collapsed_slice_dims=(0,),
                                    start_index_map=(0,))

    def _g(vv, idx):
        return lax.gather(vv, idx, dn, slice_sizes=(1,),
                          mode=lax.GatherScatterMode.PROMISE_IN_BOUNDS)

    perms = [(lax.iota(i32, 16) ^ sh)[:, None] for sh in (8, 4, 2, 1)]
    consts = [jnp.full((16, 1), e, i32) for e in range(16)]

    def _lanesum(vv):
        for p in perms:
            vv = vv + _g(vv, p)
        return vv

    def _pair(buf, r, j):
        iv = plsc.bitcast(buf[r, pl.ds(j * 32, 32)], i32)
        hi = plsc.bitcast(iv & himask, f32)
        lo = plsc.bitcast(iv << 16, f32)
        return hi, lo

    def zrow(r, carry):
        for j in range(8):
            zbuf[r, pl.ds(j * 16, 16)] = zerov
        return carry
    lax.fori_loop(0, ZROWS, zrow, 0)

    def zden(r, carry):
        for j in range(8):
            denv[r, pl.ds(j * 16, 16)] = zerov
        return carry
    lax.fori_loop(0, DROWS, zden, 0)

    for t in range(ROWS_PER_SUB // ZROWS):
        base = sid * ROWS_PER_SUB + t * ZROWS
        pltpu.sync_copy(zbuf, num_sh.at[pl.ds(base, ZROWS)])
    plsc.subcore_barrier()

    def _issue(row, qab, kvb, sq, sk):
        pltpu.async_copy(qs_hbm.at[dstb.at[row]], qab, sq)
        pltpu.async_copy(kv_hbm.at[srcb.at[row]], kvb, sk)

    def _wait(row, qab, kvb, sq, sk):
        pltpu.make_async_copy(qs_hbm.at[dstb.at[row]], qab, sq).wait()
        pltpu.make_async_copy(kv_hbm.at[srcb.at[row]], kvb, sk).wait()

    def _compute(row, qab, kvb):
        def group(g, carry2):
            dv = dstb[row, pl.ds(g * 16, 16)]
            for e in range(16):
                r = g * 16 + e
                qh, ql = _pair(qab, r, 0)
                kh, kl = _pair(kvb, r, 0)
                acc = qh * kh + ql * kl
                for j in range(1, 4):
                    qh, ql = _pair(qab, r, j)
                    kh, kl = _pair(kvb, r, j)
                    acc = acc + qh * kh
                    acc = acc + ql * kl
                exv = jnp.exp(_lanesum(acc))
                for j in range(4):
                    vh, vl = _pair(kvb, r, 4 + j)
                    wvb[r, pl.ds(j * 32, 16)] = exv * vl
                    wvb[r, pl.ds(j * 32 + 16, 16)] = exv * vh
                idx = _g(dv, consts[e])
                plsc.addupdate_scatter(denv, [idx >> 7, idx & 127], exv,
                                       mask=lane0)
            return carry2
        lax.fori_loop(0, H // 16, group, 0)
        pltpu.sync_copy(wvb, num_sh.at[dstb.at[row]], add=True)

    set0 = (qab0, kvb0, sq0, sk0)
    set1 = (qab1, kvb1, sq1, sk1)

    def block(blk, carry0):
        pltpu.sync_copy(src_hbm.at[wid, blk], srcb)
        pltpu.sync_copy(dst_hbm.at[wid, blk], dstb)

        _issue(0, *set0)

        def pair(p, carry):
            r0 = 2 * p
            _wait(r0, *set0)
            _issue(r0 + 1, *set1)
            _compute(r0, qab0, kvb0)
            _wait(r0 + 1, *set1)
            _issue(r0 + 2, *set0)
            _compute(r0 + 1, qab1, kvb1)
            return carry
        lax.fori_loop(0, IBLK - 1, pair, 0)

        r0 = IROWS - 2
        _wait(r0, *set0)
        _issue(r0 + 1, *set1)
        _compute(r0, qab0, kvb0)
        _wait(r0 + 1, *set1)
        _compute(r0 + 1, qab1, kvb1)
        return carry0
    lax.fori_loop(0, NBLK, block, 0)

    plsc.subcore_barrier()
    for t in range(ROWS_PER_SUB // ZROWS):
        base = sid * ROWS_PER_SUB + t * ZROWS
        pltpu.sync_copy(num_sh.at[pl.ds(base, ZROWS)], zbuf)
        pltpu.sync_copy(zbuf, num_out.at[cid, pl.ds(base, ZROWS)])
    pltpu.sync_copy(denv, den_out.at[wid])


@functools.lru_cache(maxsize=1)
def _make_edge_call():
  return pl.kernel(
    _edge_body,
    out_type=(jax.ShapeDtypeStruct((NC, NPAD, C), f32),
              jax.ShapeDtypeStruct((NW, DROWS, C), f32)),
    mesh=plsc.VectorSubcoreMesh(core_axis_name="c", subcore_axis_name="s",
                                num_cores=NC, num_subcores=NS),
    compiler_params=pltpu.CompilerParams(needs_layout_passes=False,
                                         use_tc_tiling_on_sc=False),
    scratch_types=(
        pltpu.VMEM((H, C), bf16),
        pltpu.VMEM((H, C), bf16),
        pltpu.VMEM((H, 2 * C), bf16),
        pltpu.VMEM((H, 2 * C), bf16),
        pltpu.VMEM((H, C), f32),
        pltpu.VMEM((IROWS, H), i32),
        pltpu.VMEM((IROWS, H), i32),
        pltpu.VMEM((ZROWS, C), f32),
        pltpu.VMEM((DROWS, C), f32),
        pltpu.VMEM_SHARED((NPAD, C), f32),
        pltpu.SemaphoreType.DMA,
        pltpu.SemaphoreType.DMA,
        pltpu.SemaphoreType.DMA,
        pltpu.SemaphoreType.DMA,
    ),
  )


def _pre_body(x_ref, pos_ref, wq, wk, wv, wp, bq, bk, bv,
              qs_out, kt_out, v_out):
    x = x_ref[...]
    scale = 1.0 / jnp.sqrt(jnp.asarray(C, f32))
    q = jnp.dot(x, wq[...], preferred_element_type=f32) + bq[...]
    k = jnp.dot(x, wk[...], preferred_element_type=f32) + bk[...]
    v = jnp.dot(x, wv[...], preferred_element_type=f32) + bv[...]
    qs_out[...] = (q * scale).astype(bf16)
    kt_out[...] = (k + jnp.dot(pos_ref[...], wp[...],
                               preferred_element_type=f32)).astype(bf16)
    v_out[...] = v.astype(bf16)


_pre_call = pl.pallas_call(
    _pre_body,
    out_shape=(jax.ShapeDtypeStruct((N, C), bf16),
               jax.ShapeDtypeStruct((N, C), bf16),
               jax.ShapeDtypeStruct((N, C), bf16)),
)


def _bn(h, g, b):
    mu = jnp.mean(h, axis=0, keepdims=True)
    var = jnp.mean((h - mu) ** 2, axis=0, keepdims=True)
    return g * (h - mu) / jnp.sqrt(var + 1e-5) + b


def _post_body(x_ref, num_ref, den_ref, wo, bo, w1, b1, w2, b2,
               na_g, na_b, bn_g, bn_b, nm_g, nm_b, x2_out):
    x = x_ref[...]
    num = num_ref[0, 0:N, :] + num_ref[1, 0:N, :]
    den = den_ref[...]
    agg = jnp.where(den > 0, num / jnp.where(den > 0, den, 1.0), 0.0)
    y = jnp.dot(agg, wo[...], preferred_element_type=f32) + bo[...]
    x1 = _bn(x + y, na_g[...], na_b[...])
    h = jnp.dot(x1, w1[...], preferred_element_type=f32) + b1[...]
    h = _bn(h, bn_g[...], bn_b[...])
    h = jnp.where(h > 0, h, 0.01 * h)
    h = jnp.dot(h, w2[...], preferred_element_type=f32) + b2[...]
    x2_out[...] = _bn(x1 + h, nm_g[...], nm_b[...])


_post_call = pl.pallas_call(
    _post_body,
    out_shape=jax.ShapeDtypeStruct((N, C), f32),
)


def kernel(x, pos, edge_index, Wq, Wk, Wv, Wo, Wp, W1, W2, bq, bk, bv, bo,
           bp, b1, b2, na_b, bn_b, nm_b, na_g, bn_g, nm_g):
    src = jnp.concatenate([edge_index[0], jnp.zeros((EPAD,), i32)])
    dst = jnp.concatenate([edge_index[1], jnp.full((EPAD,), DUMP, i32)])
    src = src.reshape(NW, NBLK, IROWS, H)
    dst = dst.reshape(NW, NBLK, IROWS, H)

    def layer(xc, p):
        (wq, wk, wv, wo, wp, w1, w2, pbq, pbk, pbv, pbo, pb1, pb2,
         pnag, pnab, pbng, pbnb, pnmg, pnmb) = p
        qs, kt, v = _pre_call(xc, pos, wq, wk, wv, wp,
                              pbq.reshape(1, C), pbk.reshape(1, C),
                              pbv.reshape(1, C))
        vp = v.reshape(N, 4, 2, 16).transpose(0, 1, 3, 2).reshape(N, C)
        kv = jnp.concatenate([kt, vp], axis=1)
        num2, den2 = _make_edge_call()(qs, kv, src, dst)
        den_node = den2.sum(axis=0).reshape(DROWS * C)[0:N, None]
        xn = _post_call(xc, num2, den_node, wo, pbo.reshape(1, C),
                        w1, pb1.reshape(1, C), w2, pb2.reshape(1, C),
                        pnag.reshape(1, C), pnab.reshape(1, C),
                        pbng.reshape(1, C), pbnb.reshape(1, C),
                        pnmg.reshape(1, C), pnmb.reshape(1, C))
        return xn, None

    x, _ = jax.lax.scan(layer, x, (Wq, Wk, Wv, Wo, Wp, W1, W2, bq, bk, bv,
                                   bo, b1, b2, na_g, na_b, bn_g, bn_b,
                                   nm_g, nm_b))
    return x


# speedup vs baseline: 1.0346x; 1.0346x over previous
import functools

import jax
import jax.numpy as jnp
from jax import lax
from jax.experimental import pallas as pl
from jax.experimental.pallas import tpu as pltpu
from jax.experimental.pallas import tpu_sc as plsc

N = 10000
E = 320000
C = 128
P = 16

NC = 2
NS = 16
NW = NC * NS
H = 64
EPW = 10240
NCHUNK = EPW // (2 * H)
IBLK = 5
NBLK = NCHUNK // IBLK
IROWS = 2 * IBLK
EPAD = NW * EPW - E
NPAD = 10240
DUMP = 10100
ROWS_PER_SUB = NPAD // NS
ZROWS = 16
DROWS = 79

f32 = jnp.float32
bf16 = jnp.bfloat16
i32 = jnp.int32


def _edge_body(qs_hbm, kt_hbm, v_hbm, src_hbm, dst_hbm,
               num_out, den_out,
               qab0, qab1, kab0, kab1, vb0, vb1, srcb, dstb, zbuf, denv,
               num_sh, sq0, sq1, sk0, sk1, sv0, sv1):
    cid = lax.axis_index("c")
    sid = lax.axis_index("s")
    wid = sid * NC + cid

    zerov = jnp.zeros((16,), f32)
    lane0 = lax.iota(i32, 16) == 0
    himask = jnp.full((16,), -65536, i32)
    dn = lax.GatherDimensionNumbers(offset_dims=(), collapsed_slice_dims=(0,),
                                    start_index_map=(0,))

    def _g(vv, idx):
        return lax.gather(vv, idx, dn, slice_sizes=(1,),
                          mode=lax.GatherScatterMode.PROMISE_IN_BOUNDS)

    perms = [(lax.iota(i32, 16) ^ sh)[:, None] for sh in (8, 4, 2, 1)]
    consts = [jnp.full((16, 1), e, i32) for e in range(16)]

    def _lanesum(vv):
        for p in perms:
            vv = vv + _g(vv, p)
        return vv

    def _pair(buf, r, j):
        iv = plsc.bitcast(buf[r, pl.ds(j * 32, 32)], i32)
        hi = plsc.bitcast(iv & himask, f32)
        lo = plsc.bitcast(iv << 16, f32)
        return hi, lo

    def zrow(r, carry):
        for j in range(8):
            zbuf[r, pl.ds(j * 16, 16)] = zerov
        return carry
    lax.fori_loop(0, ZROWS, zrow, 0)

    def zden(r, carry):
        for j in range(8):
            denv[r, pl.ds(j * 16, 16)] = zerov
        return carry
    lax.fori_loop(0, DROWS, zden, 0)

    for t in range(ROWS_PER_SUB // ZROWS):
        base = sid * ROWS_PER_SUB + t * ZROWS
        pltpu.sync_copy(zbuf, num_sh.at[pl.ds(base, ZROWS)])
    plsc.subcore_barrier()

    def _issue(row, qab, kab, vb, sq, sk, sv):
        pltpu.async_copy(qs_hbm.at[dstb.at[row]], qab, sq)
        pltpu.async_copy(kt_hbm.at[srcb.at[row]], kab, sk)
        pltpu.async_copy(v_hbm.at[srcb.at[row]], vb, sv)

    def _wait(row, qab, kab, vb, sq, sk, sv):
        pltpu.make_async_copy(qs_hbm.at[dstb.at[row]], qab, sq).wait()
        pltpu.make_async_copy(kt_hbm.at[srcb.at[row]], kab, sk).wait()
        pltpu.make_async_copy(v_hbm.at[srcb.at[row]], vb, sv).wait()

    def _compute(row, qab, kab, vb):
        def group(g, carry2):
            dv = dstb[row, pl.ds(g * 16, 16)]
            for e in range(16):
                r = g * 16 + e
                qh, ql = _pair(qab, r, 0)
                kh, kl = _pair(kab, r, 0)
                acc = qh * kh + ql * kl
                for j in range(1, 4):
                    qh, ql = _pair(qab, r, j)
                    kh, kl = _pair(kab, r, j)
                    acc = acc + qh * kh
                    acc = acc + ql * kl
                exv = jnp.exp(_lanesum(acc))
                for j in range(8):
                    vb[r, pl.ds(j * 16, 16)] = exv * vb[r, pl.ds(j * 16, 16)]
                idx = _g(dv, consts[e])
                plsc.addupdate_scatter(denv, [idx >> 7, idx & 127], exv,
                                       mask=lane0)
            return carry2
        lax.fori_loop(0, H // 16, group, 0)
        pltpu.sync_copy(vb, num_sh.at[dstb.at[row]], add=True)

    set0 = (qab0, kab0, vb0, sq0, sk0, sv0)
    set1 = (qab1, kab1, vb1, sq1, sk1, sv1)

    def block(blk, carry0):
        pltpu.sync_copy(src_hbm.at[wid, blk], srcb)
        pltpu.sync_copy(dst_hbm.at[wid, blk], dstb)

        _issue(0, *set0)

        def pair(p, carry):
            r0 = 2 * p
            _wait(r0, *set0)
            _issue(r0 + 1, *set1)
            _compute(r0, qab0, kab0, vb0)
            _wait(r0 + 1, *set1)
            _issue(r0 + 2, *set0)
            _compute(r0 + 1, qab1, kab1, vb1)
            return carry
        lax.fori_loop(0, IBLK - 1, pair, 0)

        r0 = IROWS - 2
        _wait(r0, *set0)
        _issue(r0 + 1, *set1)
        _compute(r0, qab0, kab0, vb0)
        _wait(r0 + 1, *set1)
        _compute(r0 + 1, qab1, kab1, vb1)
        return carry0
    lax.fori_loop(0, NBLK, block, 0)

    plsc.subcore_barrier()
    for t in range(ROWS_PER_SUB // ZROWS):
        base = sid * ROWS_PER_SUB + t * ZROWS
        pltpu.sync_copy(num_sh.at[pl.ds(base, ZROWS)], zbuf)
        pltpu.sync_copy(zbuf, num_out.at[cid, pl.ds(base, ZROWS)])
    pltpu.sync_copy(denv, den_out.at[wid])


@functools.lru_cache(maxsize=1)
def _make_edge_call():
  return pl.kernel(
    _edge_body,
    out_type=(jax.ShapeDtypeStruct((NC, NPAD, C), f32),
              jax.ShapeDtypeStruct((NW, DROWS, C), f32)),
    mesh=plsc.VectorSubcoreMesh(core_axis_name="c", subcore_axis_name="s",
                                num_cores=NC, num_subcores=NS),
    compiler_params=pltpu.CompilerParams(needs_layout_passes=False,
                                         use_tc_tiling_on_sc=False),
    scratch_types=(
        pltpu.VMEM((H, C), bf16),
        pltpu.VMEM((H, C), bf16),
        pltpu.VMEM((H, C), bf16),
        pltpu.VMEM((H, C), bf16),
        pltpu.VMEM((H, C), f32),
        pltpu.VMEM((H, C), f32),
        pltpu.VMEM((IROWS, H), i32),
        pltpu.VMEM((IROWS, H), i32),
        pltpu.VMEM((ZROWS, C), f32),
        pltpu.VMEM((DROWS, C), f32),
        pltpu.VMEM_SHARED((NPAD, C), f32),
        pltpu.SemaphoreType.DMA,
        pltpu.SemaphoreType.DMA,
        pltpu.SemaphoreType.DMA,
        pltpu.SemaphoreType.DMA,
        pltpu.SemaphoreType.DMA,
        pltpu.SemaphoreType.DMA,
    ),
  )


def _pre_body(x_ref, pos_ref, wq, wk, wv, wp, bq, bk, bv,
              qs_out, kt_out, v_out):
    x = x_ref[...]
    scale = 1.0 / jnp.sqrt(jnp.asarray(C, f32))
    q = jnp.dot(x, wq[...], preferred_element_type=f32) + bq[...]
    k = jnp.dot(x, wk[...], preferred_element_type=f32) + bk[...]
    v = jnp.dot(x, wv[...], preferred_element_type=f32) + bv[...]
    qs_out[...] = (q * scale).astype(bf16)
    kt_out[...] = (k + jnp.dot(pos_ref[...], wp[...],
                               preferred_element_type=f32)).astype(bf16)
    v_out[...] = v


_pre_call = pl.pallas_call(
    _pre_body,
    out_shape=(jax.ShapeDtypeStruct((N, C), bf16),
               jax.ShapeDtypeStruct((N, C), bf16),
               jax.ShapeDtypeStruct((N, C), f32)),
)


def _bn(h, g, b):
    mu = jnp.mean(h, axis=0, keepdims=True)
    var = jnp.mean((h - mu) ** 2, axis=0, keepdims=True)
    return g * (h - mu) / jnp.sqrt(var + 1e-5) + b


def _post_body(x_ref, num_ref, den_ref, wo, bo, w1, b1, w2, b2,
               na_g, na_b, bn_g, bn_b, nm_g, nm_b, x2_out):
    x = x_ref[...]
    num = num_ref[0, 0:N, :] + num_ref[1, 0:N, :]
    den = den_ref[...]
    agg = jnp.where(den > 0, num / jnp.where(den > 0, den, 1.0), 0.0)
    y = jnp.dot(agg, wo[...], preferred_element_type=f32) + bo[...]
    x1 = _bn(x + y, na_g[...], na_b[...])
    h = jnp.dot(x1, w1[...], preferred_element_type=f32) + b1[...]
    h = _bn(h, bn_g[...], bn_b[...])
    h = jnp.where(h > 0, h, 0.01 * h)
    h = jnp.dot(h, w2[...], preferred_element_type=f32) + b2[...]
    x2_out[...] = _bn(x1 + h, nm_g[...], nm_b[...])


_post_call = pl.pallas_call(
    _post_body,
    out_shape=jax.ShapeDtypeStruct((N, C), f32),
)


def kernel(x, pos, edge_index, Wq, Wk, Wv, Wo, Wp, W1, W2, bq, bk, bv, bo,
           bp, b1, b2, na_b, bn_b, nm_b, na_g, bn_g, nm_g):
    src = jnp.concatenate([edge_index[0], jnp.zeros((EPAD,), i32)])
    dst = jnp.concatenate([edge_index[1], jnp.full((EPAD,), DUMP, i32)])
    src = src.reshape(NW, NBLK, IROWS, H)
    dst = dst.reshape(NW, NBLK, IROWS, H)

    def layer(xc, p):
        (wq, wk, wv, wo, wp, w1, w2, pbq, pbk, pbv, pbo, pb1, pb2,
         pnag, pnab, pbng, pbnb, pnmg, pnmb) = p
        qs, kt, v = _pre_call(xc, pos, wq, wk, wv, wp,
                              pbq.reshape(1, C), pbk.reshape(1, C),
                              pbv.reshape(1, C))
        num2, den2 = _make_edge_call()(qs, kt, v, src, dst)
        den_node = den2.sum(axis=0).reshape(DROWS * C)[0:N, None]
        xn = _post_call(xc, num2, den_node, wo, pbo.reshape(1, C),
                        w1, pb1.reshape(1, C), w2, pb2.reshape(1, C),
                        pnag.reshape(1, C), pnab.reshape(1, C),
                        pbng.reshape(1, C), pbnb.reshape(1, C),
                        pnmg.reshape(1, C), pnmb.reshape(1, C))
        return xn, None

    x, _ = jax.lax.scan(layer, x, (Wq, Wk, Wv, Wo, Wp, W1, W2, bq, bk, bv,
                                   bo, b1, b2, na_g, na_b, bn_g, bn_b,
                                   nm_g, nm_b))
    return x


# speedup vs baseline: 1.1266x; 1.0889x over previous
import functools

import jax
import jax.numpy as jnp
from jax import lax
from jax.experimental import pallas as pl
from jax.experimental.pallas import tpu as pltpu
from jax.experimental.pallas import tpu_sc as plsc

N = 10000
E = 320000
C = 128
P = 16

NC = 2
NS = 16
NW = NC * NS
H = 48
EPW = 10080
NCHUNK = EPW // (2 * H)
IBLK = 15
NBLK = NCHUNK // IBLK
IROWS = 2 * IBLK
EPAD = NW * EPW - E
NPAD = 10240
DUMP = 10100
ROWS_PER_SUB = NPAD // NS
ZROWS = 32
DROWS = 79

f32 = jnp.float32
bf16 = jnp.bfloat16
i32 = jnp.int32


def _edge_body(qs_hbm, kt_hbm, v_hbm, src_hbm, dst_hbm,
               num_out, den_out,
               qab0, qab1, kab0, kab1, vb0, vb1, srcb, dstb, zbuf, denv,
               num_sh, sq0, sq1, sk0, sk1, sv0, sv1):
    cid = lax.axis_index("c")
    sid = lax.axis_index("s")
    wid = sid * NC + cid

    zerov = jnp.zeros((16,), f32)
    lane0 = lax.iota(i32, 16) == 0
    himask = jnp.full((16,), -65536, i32)
    dn = lax.GatherDimensionNumbers(offset_dims=(), collapsed_slice_dims=(0,),
                                    start_index_map=(0,))

    def _g(vv, idx):
        return lax.gather(vv, idx, dn, slice_sizes=(1,),
                          mode=lax.GatherScatterMode.PROMISE_IN_BOUNDS)

    perms = [(lax.iota(i32, 16) ^ sh)[:, None] for sh in (8, 4, 2, 1)]
    consts = [jnp.full((16, 1), e, i32) for e in range(16)]

    def _lanesum(vv):
        for p in perms:
            vv = vv + _g(vv, p)
        return vv

    def _pair(buf, r, j):
        iv = plsc.bitcast(buf[r, pl.ds(j * 32, 32)], i32)
        hi = plsc.bitcast(iv & himask, f32)
        lo = plsc.bitcast(iv << 16, f32)
        return hi, lo

    def zrow(r, carry):
        for j in range(8):
            zbuf[r, pl.ds(j * 16, 16)] = zerov
        return carry
    lax.fori_loop(0, ZROWS, zrow, 0)

    def zden(r, carry):
        for j in range(8):
            denv[r, pl.ds(j * 16, 16)] = zerov
        return carry
    lax.fori_loop(0, DROWS, zden, 0)

    for t in range(ROWS_PER_SUB // ZROWS):
        base = sid * ROWS_PER_SUB + t * ZROWS
        pltpu.sync_copy(zbuf, num_sh.at[pl.ds(base, ZROWS)])
    plsc.subcore_barrier()

    def _issue(row, qab, kab, vb, sq, sk, sv):
        pltpu.async_copy(qs_hbm.at[dstb.at[row]], qab, sq)
        pltpu.async_copy(kt_hbm.at[srcb.at[row]], kab, sk)
        pltpu.async_copy(v_hbm.at[srcb.at[row]], vb, sv)

    def _wait(row, qab, kab, vb, sq, sk, sv):
        pltpu.make_async_copy(qs_hbm.at[dstb.at[row]], qab, sq).wait()
        pltpu.make_async_copy(kt_hbm.at[srcb.at[row]], kab, sk).wait()
        pltpu.make_async_copy(v_hbm.at[srcb.at[row]], vb, sv).wait()

    def _compute(row, qab, kab, vb):
        def group(g, carry2):
            dv = dstb[row, pl.ds(g * 16, 16)]
            for e in range(16):
                r = g * 16 + e
                qh, ql = _pair(qab, r, 0)
                kh, kl = _pair(kab, r, 0)
                acc = qh * kh + ql * kl
                for j in range(1, 4):
                    qh, ql = _pair(qab, r, j)
                    kh, kl = _pair(kab, r, j)
                    acc = acc + qh * kh
                    acc = acc + ql * kl
                exv = jnp.exp(_lanesum(acc))
                for j in range(8):
                    vb[r, pl.ds(j * 16, 16)] = exv * vb[r, pl.ds(j * 16, 16)]
                idx = _g(dv, consts[e])
                plsc.addupdate_scatter(denv, [idx >> 7, idx & 127], exv,
                                       mask=lane0)
            return carry2
        lax.fori_loop(0, H // 16, group, 0)
        pltpu.sync_copy(vb, num_sh.at[dstb.at[row]], add=True)

    set0 = (qab0, kab0, vb0, sq0, sk0, sv0)
    set1 = (qab1, kab1, vb1, sq1, sk1, sv1)

    def block(blk, carry0):
        pltpu.sync_copy(src_hbm.at[wid, blk], srcb)
        pltpu.sync_copy(dst_hbm.at[wid, blk], dstb)

        _issue(0, *set0)

        def pair(p, carry):
            r0 = 2 * p
            _wait(r0, *set0)
            _issue(r0 + 1, *set1)
            _compute(r0, qab0, kab0, vb0)
            _wait(r0 + 1, *set1)
            _issue(r0 + 2, *set0)
            _compute(r0 + 1, qab1, kab1, vb1)
            return carry
        lax.fori_loop(0, IBLK - 1, pair, 0)

        r0 = IROWS - 2
        _wait(r0, *set0)
        _issue(r0 + 1, *set1)
        _compute(r0, qab0, kab0, vb0)
        _wait(r0 + 1, *set1)
        _compute(r0 + 1, qab1, kab1, vb1)
        return carry0
    lax.fori_loop(0, NBLK, block, 0)

    plsc.subcore_barrier()
    for t in range(ROWS_PER_SUB // ZROWS):
        base = sid * ROWS_PER_SUB + t * ZROWS
        pltpu.sync_copy(num_sh.at[pl.ds(base, ZROWS)], zbuf)
        pltpu.sync_copy(zbuf, num_out.at[cid, pl.ds(base, ZROWS)])
    pltpu.sync_copy(denv, den_out.at[wid])


@functools.lru_cache(maxsize=1)
def _make_edge_call():
  return pl.kernel(
    _edge_body,
    out_type=(jax.ShapeDtypeStruct((NC, NPAD, C), f32),
              jax.ShapeDtypeStruct((NW, DROWS, C), f32)),
    mesh=plsc.VectorSubcoreMesh(core_axis_name="c", subcore_axis_name="s",
                                num_cores=NC, num_subcores=NS),
    compiler_params=pltpu.CompilerParams(needs_layout_passes=False,
                                         use_tc_tiling_on_sc=False),
    scratch_types=(
        pltpu.VMEM((H, C), bf16),
        pltpu.VMEM((H, C), bf16),
        pltpu.VMEM((H, C), bf16),
        pltpu.VMEM((H, C), bf16),
        pltpu.VMEM((H, C), f32),
        pltpu.VMEM((H, C), f32),
        pltpu.VMEM((IROWS, H), i32),
        pltpu.VMEM((IROWS, H), i32),
        pltpu.VMEM((ZROWS, C), f32),
        pltpu.VMEM((DROWS, C), f32),
        pltpu.VMEM_SHARED((NPAD, C), f32),
        pltpu.SemaphoreType.DMA,
        pltpu.SemaphoreType.DMA,
        pltpu.SemaphoreType.DMA,
        pltpu.SemaphoreType.DMA,
        pltpu.SemaphoreType.DMA,
        pltpu.SemaphoreType.DMA,
    ),
  )


def _pre_body(x_ref, pos_ref, wq, wk, wv, wp, bq, bk, bv,
              qs_out, kt_out, v_out):
    x = x_ref[...]
    scale = 1.0 / jnp.sqrt(jnp.asarray(C, f32))
    q = jnp.dot(x, wq[...], preferred_element_type=f32) + bq[...]
    k = jnp.dot(x, wk[...], preferred_element_type=f32) + bk[...]
    v = jnp.dot(x, wv[...], preferred_element_type=f32) + bv[...]
    qs_out[...] = (q * scale).astype(bf16)
    kt_out[...] = (k + jnp.dot(pos_ref[...], wp[...],
                               preferred_element_type=f32)).astype(bf16)
    v_out[...] = v


_pre_call = pl.pallas_call(
    _pre_body,
    out_shape=(jax.ShapeDtypeStruct((N, C), bf16),
               jax.ShapeDtypeStruct((N, C), bf16),
               jax.ShapeDtypeStruct((N, C), f32)),
)


def _bn(h, g, b):
    mu = jnp.mean(h, axis=0, keepdims=True)
    var = jnp.mean((h - mu) ** 2, axis=0, keepdims=True)
    return g * (h - mu) / jnp.sqrt(var + 1e-5) + b


def _post_body(x_ref, num_ref, den_ref, wo, bo, w1, b1, w2, b2,
               na_g, na_b, bn_g, bn_b, nm_g, nm_b, x2_out):
    x = x_ref[...]
    num = num_ref[0, 0:N, :] + num_ref[1, 0:N, :]
    den = den_ref[...]
    agg = jnp.where(den > 0, num / jnp.where(den > 0, den, 1.0), 0.0)
    y = jnp.dot(agg, wo[...], preferred_element_type=f32) + bo[...]
    x1 = _bn(x + y, na_g[...], na_b[...])
    h = jnp.dot(x1, w1[...], preferred_element_type=f32) + b1[...]
    h = _bn(h, bn_g[...], bn_b[...])
    h = jnp.where(h > 0, h, 0.01 * h)
    h = jnp.dot(h, w2[...], preferred_element_type=f32) + b2[...]
    x2_out[...] = _bn(x1 + h, nm_g[...], nm_b[...])


_post_call = pl.pallas_call(
    _post_body,
    out_shape=jax.ShapeDtypeStruct((N, C), f32),
)


def kernel(x, pos, edge_index, Wq, Wk, Wv, Wo, Wp, W1, W2, bq, bk, bv, bo,
           bp, b1, b2, na_b, bn_b, nm_b, na_g, bn_g, nm_g):
    src = jnp.concatenate([edge_index[0], jnp.zeros((EPAD,), i32)])
    dst = jnp.concatenate([edge_index[1], jnp.full((EPAD,), DUMP, i32)])
    src = src.reshape(NW, NBLK, IROWS, H)
    dst = dst.reshape(NW, NBLK, IROWS, H)

    def layer(xc, p):
        (wq, wk, wv, wo, wp, w1, w2, pbq, pbk, pbv, pbo, pb1, pb2,
         pnag, pnab, pbng, pbnb, pnmg, pnmb) = p
        qs, kt, v = _pre_call(xc, pos, wq, wk, wv, wp,
                              pbq.reshape(1, C), pbk.reshape(1, C),
                              pbv.reshape(1, C))
        num2, den2 = _make_edge_call()(qs, kt, v, src, dst)
        den_node = den2.sum(axis=0).reshape(DROWS * C)[0:N, None]
        xn = _post_call(xc, num2, den_node, wo, pbo.reshape(1, C),
                        w1, pb1.reshape(1, C), w2, pb2.reshape(1, C),
                        pnag.reshape(1, C), pnab.reshape(1, C),
                        pbng.reshape(1, C), pbnb.reshape(1, C),
                        pnmg.reshape(1, C), pnmb.reshape(1, C))
        return xn, None

    x, _ = jax.lax.scan(layer, x, (Wq, Wk, Wv, Wo, Wp, W1, W2, bq, bk, bv,
                                   bo, b1, b2, na_g, na_b, bn_g, bn_b,
                                   nm_g, nm_b))
    return x


# speedup vs baseline: 1.2206x; 1.0834x over previous
import functools

import jax
import jax.numpy as jnp
from jax import lax
from jax.experimental import pallas as pl
from jax.experimental.pallas import tpu as pltpu
from jax.experimental.pallas import tpu_sc as plsc

N = 10000
E = 320000
C = 128
P = 16

NC = 2
NS = 16
NW = NC * NS
H = 48
EPW = 10080
NCHUNK = EPW // (2 * H)
IBLK = 15
NBLK = NCHUNK // IBLK
IROWS = 2 * IBLK
EPAD = NW * EPW - E
NPAD = 10240
DUMP = NPAD - 1
ROWS_PER_SUB = NPAD // NS
ZROWS = 32
DROWS = 80

f32 = jnp.float32
bf16 = jnp.bfloat16
i32 = jnp.int32


def _edge_body(qs_hbm, kt_hbm, v_hbm, src_hbm, dst_hbm,
               num_out, den_out,
               qab0, qab1, kab0, kab1, vb0, vb1, srcb, dstb, zbuf, denv,
               num_sh, sq0, sq1, sk0, sk1, sv0, sv1):
    cid = lax.axis_index("c")
    sid = lax.axis_index("s")
    wid = sid * NC + cid

    zerov = jnp.zeros((16,), f32)
    lane0 = lax.iota(i32, 16) == 0
    himask = jnp.full((16,), -65536, i32)
    dn = lax.GatherDimensionNumbers(offset_dims=(), collapsed_slice_dims=(0,),
                                    start_index_map=(0,))

    def _g(vv, idx):
        return lax.gather(vv, idx, dn, slice_sizes=(1,),
                          mode=lax.GatherScatterMode.PROMISE_IN_BOUNDS)

    perms = [(lax.iota(i32, 16) ^ sh)[:, None] for sh in (8, 4, 2, 1)]
    consts = [jnp.full((16, 1), e, i32) for e in range(16)]

    def _lanesum(vv):
        for p in perms:
            vv = vv + _g(vv, p)
        return vv

    def _pair(buf, r, j):
        iv = plsc.bitcast(buf[r, pl.ds(j * 32, 32)], i32)
        hi = plsc.bitcast(iv & himask, f32)
        lo = plsc.bitcast(iv << 16, f32)
        return hi, lo

    def zrow(r, carry):
        for j in range(8):
            zbuf[r, pl.ds(j * 16, 16)] = zerov
        return carry
    lax.fori_loop(0, ZROWS, zrow, 0)

    def zden(r, carry):
        for j in range(8):
            denv[r, pl.ds(j * 16, 16)] = zerov
        return carry
    lax.fori_loop(0, DROWS, zden, 0)

    for t in range(ROWS_PER_SUB // ZROWS):
        base = sid * ROWS_PER_SUB + t * ZROWS
        pltpu.sync_copy(zbuf, num_sh.at[pl.ds(base, ZROWS)])
    plsc.subcore_barrier()

    def _issue(row, qab, kab, vb, sq, sk, sv):
        pltpu.async_copy(qs_hbm.at[dstb.at[row]], qab, sq)
        pltpu.async_copy(kt_hbm.at[srcb.at[row]], kab, sk)
        pltpu.async_copy(v_hbm.at[srcb.at[row]], vb, sv)

    def _wait(row, qab, kab, vb, sq, sk, sv):
        pltpu.make_async_copy(qs_hbm.at[dstb.at[row]], qab, sq).wait()
        pltpu.make_async_copy(kt_hbm.at[srcb.at[row]], kab, sk).wait()
        pltpu.make_async_copy(v_hbm.at[srcb.at[row]], vb, sv).wait()

    def _compute(row, qab, kab, vb):
        def group(g, carry2):
            dv = dstb[row, pl.ds(g * 16, 16)]
            for e in range(16):
                r = g * 16 + e
                qh, ql = _pair(qab, r, 0)
                kh, kl = _pair(kab, r, 0)
                acc = qh * kh + ql * kl
                for j in range(1, 4):
                    qh, ql = _pair(qab, r, j)
                    kh, kl = _pair(kab, r, j)
                    acc = acc + qh * kh
                    acc = acc + ql * kl
                exv = jnp.exp(_lanesum(acc))
                for j in range(8):
                    vb[r, pl.ds(j * 16, 16)] = exv * vb[r, pl.ds(j * 16, 16)]
                idx = _g(dv, consts[e])
                plsc.addupdate_scatter(denv, [idx >> 7, idx & 127], exv,
                                       mask=lane0)
            return carry2
        lax.fori_loop(0, H // 16, group, 0)
        pltpu.sync_copy(vb, num_sh.at[dstb.at[row]], add=True)

    set0 = (qab0, kab0, vb0, sq0, sk0, sv0)
    set1 = (qab1, kab1, vb1, sq1, sk1, sv1)

    def block(blk, carry0):
        pltpu.sync_copy(src_hbm.at[wid, blk], srcb)
        pltpu.sync_copy(dst_hbm.at[wid, blk], dstb)

        _issue(0, *set0)

        def pair(p, carry):
            r0 = 2 * p
            _wait(r0, *set0)
            _issue(r0 + 1, *set1)
            _compute(r0, qab0, kab0, vb0)
            _wait(r0 + 1, *set1)
            _issue(r0 + 2, *set0)
            _compute(r0 + 1, qab1, kab1, vb1)
            return carry
        lax.fori_loop(0, IBLK - 1, pair, 0)

        r0 = IROWS - 2
        _wait(r0, *set0)
        _issue(r0 + 1, *set1)
        _compute(r0, qab0, kab0, vb0)
        _wait(r0 + 1, *set1)
        _compute(r0 + 1, qab1, kab1, vb1)
        return carry0
    lax.fori_loop(0, NBLK, block, 0)

    plsc.subcore_barrier()
    for t in range(ROWS_PER_SUB // ZROWS):
        base = sid * ROWS_PER_SUB + t * ZROWS
        pltpu.sync_copy(num_sh.at[pl.ds(base, ZROWS)], zbuf)
        pltpu.sync_copy(zbuf, num_out.at[cid, pl.ds(base, ZROWS)])
    pltpu.sync_copy(denv, den_out.at[wid])


@functools.lru_cache(maxsize=1)
def _make_edge_call():
  return pl.kernel(
    _edge_body,
    out_type=(jax.ShapeDtypeStruct((NC, NPAD, C), f32),
              jax.ShapeDtypeStruct((NW, DROWS, C), f32)),
    mesh=plsc.VectorSubcoreMesh(core_axis_name="c", subcore_axis_name="s",
                                num_cores=NC, num_subcores=NS),
    compiler_params=pltpu.CompilerParams(needs_layout_passes=False,
                                         use_tc_tiling_on_sc=False),
    scratch_types=(
        pltpu.VMEM((H, C), bf16),
        pltpu.VMEM((H, C), bf16),
        pltpu.VMEM((H, C), bf16),
        pltpu.VMEM((H, C), bf16),
        pltpu.VMEM((H, C), f32),
        pltpu.VMEM((H, C), f32),
        pltpu.VMEM((IROWS, H), i32),
        pltpu.VMEM((IROWS, H), i32),
        pltpu.VMEM((ZROWS, C), f32),
        pltpu.VMEM((DROWS, C), f32),
        pltpu.VMEM_SHARED((NPAD, C), f32),
        pltpu.SemaphoreType.DMA,
        pltpu.SemaphoreType.DMA,
        pltpu.SemaphoreType.DMA,
        pltpu.SemaphoreType.DMA,
        pltpu.SemaphoreType.DMA,
        pltpu.SemaphoreType.DMA,
    ),
  )


def _pre_body(x_ref, pos_ref, wq, wk, wv, wp, bq, bk, bv,
              qs_out, kt_out, v_out):
    x = x_ref[...]
    scale = 1.0 / jnp.sqrt(jnp.asarray(C, f32))
    q = jnp.dot(x, wq[...], preferred_element_type=f32) + bq[...]
    k = jnp.dot(x, wk[...], preferred_element_type=f32) + bk[...]
    v = jnp.dot(x, wv[...], preferred_element_type=f32) + bv[...]
    qs_out[...] = (q * scale).astype(bf16)
    kt_out[...] = (k + jnp.dot(pos_ref[...], wp[...],
                               preferred_element_type=f32)).astype(bf16)
    v_out[...] = v


_pre_call = pl.pallas_call(
    _pre_body,
    out_shape=(jax.ShapeDtypeStruct((N, C), bf16),
               jax.ShapeDtypeStruct((N, C), bf16),
               jax.ShapeDtypeStruct((N, C), f32)),
)


def _bn(h, g, b):
    mu = jnp.mean(h, axis=0, keepdims=True)
    var = jnp.mean((h - mu) ** 2, axis=0, keepdims=True)
    return g * (h - mu) / jnp.sqrt(var + 1e-5) + b


def _post_body(x_ref, num_ref, den_ref, wo, bo, w1, b1, w2, b2,
               na_g, na_b, bn_g, bn_b, nm_g, nm_b, x2_out):
    x = x_ref[...]
    num = num_ref[0, 0:N, :] + num_ref[1, 0:N, :]
    den = den_ref[...]
    agg = jnp.where(den > 0, num / jnp.where(den > 0, den, 1.0), 0.0)
    y = jnp.dot(agg, wo[...], preferred_element_type=f32) + bo[...]
    x1 = _bn(x + y, na_g[...], na_b[...])
    h = jnp.dot(x1, w1[...], preferred_element_type=f32) + b1[...]
    h = _bn(h, bn_g[...], bn_b[...])
    h = jnp.where(h > 0, h, 0.01 * h)
    h = jnp.dot(h, w2[...], preferred_element_type=f32) + b2[...]
    x2_out[...] = _bn(x1 + h, nm_g[...], nm_b[...])


_post_call = pl.pallas_call(
    _post_body,
    out_shape=jax.ShapeDtypeStruct((N, C), f32),
)


def kernel(x, pos, edge_index, Wq, Wk, Wv, Wo, Wp, W1, W2, bq, bk, bv, bo,
           bp, b1, b2, na_b, bn_b, nm_b, na_g, bn_g, nm_g):
    src = jnp.concatenate([edge_index[0], jnp.zeros((EPAD,), i32)])
    dst = jnp.concatenate([edge_index[1], jnp.full((EPAD,), DUMP, i32)])
    src = src.reshape(NW, NBLK, IROWS, H)
    dst = dst.reshape(NW, NBLK, IROWS, H)

    def layer(xc, p):
        (wq, wk, wv, wo, wp, w1, w2, pbq, pbk, pbv, pbo, pb1, pb2,
         pnag, pnab, pbng, pbnb, pnmg, pnmb) = p
        qs, kt, v = _pre_call(xc, pos, wq, wk, wv, wp,
                              pbq.reshape(1, C), pbk.reshape(1, C),
                              pbv.reshape(1, C))
        num2, den2 = _make_edge_call()(qs, kt, v, src, dst)
        den_node = den2.sum(axis=0).reshape(DROWS * C)[0:N, None]
        xn = _post_call(xc, num2, den_node, wo, pbo.reshape(1, C),
                        w1, pb1.reshape(1, C), w2, pb2.reshape(1, C),
                        pnag.reshape(1, C), pnab.reshape(1, C),
                        pbng.reshape(1, C), pbnb.reshape(1, C),
                        pnmg.reshape(1, C), pnmb.reshape(1, C))
        return xn, None

    x, _ = jax.lax.scan(layer, x, (Wq, Wk, Wv, Wo, Wp, W1, W2, bq, bk, bv,
                                   bo, b1, b2, na_g, na_b, bn_g, bn_b,
                                   nm_g, nm_b))
    return x
